# baseline (speedup 1.0000x reference)
import functools

import jax
import jax.numpy as jnp
from jax import lax
from jax.experimental import pallas as pl
from jax.experimental.pallas import tpu as pltpu
from jax.experimental.pallas import tpu_sc as plsc

N = 10000
E = 320000
IN_CH, HID_CH, CLS_CH = 128, 256, 64

NC, NS = 2, 16
NW = NC * NS
CH = 128
EPW = E // NW
NCHUNK = 80
EPW_PAD = NCHUNK * CH
NACC = 10240
RPT = NACC // NS

_mesh = plsc.VectorSubcoreMesh(core_axis_name="c", subcore_axis_name="s")


@functools.partial(
    pl.kernel,
    out_type=jax.ShapeDtypeStruct((NC, NACC), jnp.float32),
    mesh=_mesh,
    scratch_types=[
        pltpu.VMEM((NCHUNK, CH), jnp.int32),
        pltpu.VMEM((CH,), jnp.float32),
        pltpu.VMEM_SHARED((NACC,), jnp.float32),
    ],
)
def _sc_degree(dst_hbm, zeros_hbm, deg_hbm, idx_d, ones_v, deg_sh):
    c = lax.axis_index("c")
    s = lax.axis_index("s")
    w = s * NC + c
    pltpu.sync_copy(dst_hbm.at[w], idx_d)
    for i in range(CH // 16):
        ones_v[pl.ds(i * 16, 16)] = jnp.ones((16,), jnp.float32)
    r0 = s * RPT
    pltpu.sync_copy(zeros_hbm.at[pl.ds(r0, RPT)], deg_sh.at[pl.ds(r0, RPT)])
    plsc.subcore_barrier()

    def body(j, carry):
        pltpu.sync_copy(ones_v, deg_sh.at[idx_d.at[j]], add=True)
        return carry

    lax.fori_loop(0, NCHUNK, body, 0)
    plsc.subcore_barrier()
    pltpu.sync_copy(deg_sh.at[pl.ds(r0, RPT)], deg_hbm.at[c, pl.ds(r0, RPT)])


def _make_sc_agg(D, NBUF):
    params = None if D % 128 == 0 else pltpu.CompilerParams(use_tc_tiling_on_sc=False)
    NPJ = NCHUNK // NBUF
    assert NPJ * NBUF == NCHUNK

    @functools.partial(
        pl.kernel,
        out_type=jax.ShapeDtypeStruct((NC, NACC, D), jnp.float32),
        mesh=_mesh,
        compiler_params=params,
        scratch_types=[
            pltpu.VMEM((2, NBUF, CH), jnp.int32),
            pltpu.VMEM((2, NBUF, CH), jnp.int32),
            pltpu.VMEM_SHARED((NACC, D), jnp.float32),
        ] + [pltpu.VMEM((CH, D), jnp.float32) for _ in range(NBUF)]
          + [pltpu.SemaphoreType.DMA for _ in range(2 * NBUF + 2)],
    )
    def _sc_agg(src_hbm, dst_hbm, feat_hbm, zeros_hbm, out_hbm,
                isv, idv, acc_sh, *bufs_and_sems):
        rows = bufs_and_sems[:NBUF]
        gsem = bufs_and_sems[NBUF:2 * NBUF]
        ssem = bufs_and_sems[2 * NBUF:3 * NBUF]
        isem, dsem = bufs_and_sems[3 * NBUF:]
        c = lax.axis_index("c")
        s = lax.axis_index("s")
        w = s * NC + c
        r0 = s * RPT
        pltpu.sync_copy(zeros_hbm.at[pl.ds(r0, RPT)], acc_sh.at[pl.ds(r0, RPT)])
        pltpu.sync_copy(src_hbm.at[w, pl.ds(0, NBUF)], isv.at[0])
        pltpu.sync_copy(dst_hbm.at[w, pl.ds(0, NBUF)], idv.at[0])
        plsc.subcore_barrier()

        for k in range(NBUF):
            pltpu.async_copy(feat_hbm.at[isv.at[0, k]], rows[k], gsem[k])

        def body(j, carry):
            p = lax.rem(j, 2)
            pn = lax.rem(j + 1, 2)

            @pl.when(j < NPJ - 1)
            def _():
                nb = (j + 1) * NBUF
                pltpu.async_copy(src_hbm.at[w, pl.ds(nb, NBUF)], isv.at[pn], isem)
                pltpu.async_copy(dst_hbm.at[w, pl.ds(nb, NBUF)], idv.at[pn], dsem)

            for k in range(NBUF):
                pltpu.make_async_copy(feat_hbm.at[isv.at[p, k]], rows[k],
                                      gsem[k]).wait()
                pltpu.async_copy(rows[k], acc_sh.at[idv.at[p, k]], ssem[k],
                                 add=True)

            @pl.when(j < NPJ - 1)
            def _():
                pltpu.make_async_copy(src_hbm.at[w, pl.ds(0, NBUF)], isv.at[pn],
                                      isem).wait()
                pltpu.make_async_copy(dst_hbm.at[w, pl.ds(0, NBUF)], idv.at[pn],
                                      dsem).wait()
                for k in range(NBUF):
                    pltpu.make_async_copy(rows[k], acc_sh.at[idv.at[p, k]],
                                          ssem[k]).wait()
                    pltpu.async_copy(feat_hbm.at[isv.at[pn, k]], rows[k], gsem[k])
            return carry

        lax.fori_loop(0, NPJ, body, 0)
        pf = (NPJ - 1) % 2
        for k in range(NBUF):
            pltpu.make_async_copy(rows[k], acc_sh.at[idv.at[pf, k]],
                                  ssem[k]).wait()
        plsc.subcore_barrier()
        pltpu.sync_copy(acc_sh.at[pl.ds(r0, RPT)], out_hbm.at[c, pl.ds(r0, RPT)])

    return _sc_agg


_sc_agg_in = _make_sc_agg(IN_CH, 2)
_sc_agg_cls = _make_sc_agg(CLS_CH, 4)


_BR = 1000


def _tc_prescale_body(dega, degb, x, xp, dinv):
    di = lax.rsqrt(dega[...] + degb[...] + 1.0)
    dinv[...] = di
    xp[...] = x[...] * di


def _tc_prescale(dega, degb, x):
    grid = (N // _BR,)
    return pl.pallas_call(
        _tc_prescale_body,
        grid=grid,
        in_specs=[
            pl.BlockSpec((_BR, 1), lambda i: (i, 0)),
            pl.BlockSpec((_BR, 1), lambda i: (i, 0)),
            pl.BlockSpec((_BR, IN_CH), lambda i: (i, 0)),
        ],
        out_specs=[
            pl.BlockSpec((_BR, IN_CH), lambda i: (i, 0)),
            pl.BlockSpec((_BR, 1), lambda i: (i, 0)),
        ],
        out_shape=[
            jax.ShapeDtypeStruct((N, IN_CH), jnp.float32),
            jax.ShapeDtypeStruct((N, 1), jnp.float32),
        ],
    )(dega, degb, x)


def _tc_mid_body(z1a, z1b, xp, dinv, W1, b1, W2, y2):
    di = dinv[...]
    u = (z1a[...] + z1b[...] + xp[...]) * di
    h = jnp.dot(u, W1[...], preferred_element_type=jnp.float32) + b1[...]
    h = jnp.maximum(h, 0.0)
    g = jnp.dot(h, W2[...], preferred_element_type=jnp.float32)
    y2[...] = g * di


def _tc_mid(z1a, z1b, xp, dinv, W1, b1, W2):
    grid = (N // _BR,)
    return pl.pallas_call(
        _tc_mid_body,
        grid=grid,
        in_specs=[
            pl.BlockSpec((_BR, IN_CH), lambda i: (i, 0)),
            pl.BlockSpec((_BR, IN_CH), lambda i: (i, 0)),
            pl.BlockSpec((_BR, IN_CH), lambda i: (i, 0)),
            pl.BlockSpec((_BR, 1), lambda i: (i, 0)),
            pl.BlockSpec((IN_CH, HID_CH), lambda i: (0, 0)),
            pl.BlockSpec((1, HID_CH), lambda i: (0, 0)),
            pl.BlockSpec((HID_CH, CLS_CH), lambda i: (0, 0)),
        ],
        out_specs=pl.BlockSpec((_BR, CLS_CH), lambda i: (i, 0)),
        out_shape=jax.ShapeDtypeStruct((N, CLS_CH), jnp.float32),
    )(z1a, z1b, xp, dinv, W1, b1, W2)


def _tc_final_body(z2a, z2b, y2, dinv, b2, out):
    out[...] = (z2a[...] + z2b[...] + y2[...]) * dinv[...] + b2[...]


def _tc_final(z2a, z2b, y2, dinv, b2):
    grid = (N // _BR,)
    return pl.pallas_call(
        _tc_final_body,
        grid=grid,
        in_specs=[
            pl.BlockSpec((_BR, CLS_CH), lambda i: (i, 0)),
            pl.BlockSpec((_BR, CLS_CH), lambda i: (i, 0)),
            pl.BlockSpec((_BR, CLS_CH), lambda i: (i, 0)),
            pl.BlockSpec((_BR, 1), lambda i: (i, 0)),
            pl.BlockSpec((1, CLS_CH), lambda i: (0, 0)),
        ],
        out_specs=pl.BlockSpec((_BR, CLS_CH), lambda i: (i, 0)),
        out_shape=jax.ShapeDtypeStruct((N, CLS_CH), jnp.float32),
    )(z2a, z2b, y2, dinv, b2)


def kernel(x, edge_index, W1, b1, W2, b2):
    src = edge_index[0].astype(jnp.int32).reshape(NW, EPW)
    dst = edge_index[1].astype(jnp.int32).reshape(NW, EPW)
    pad = EPW_PAD - EPW
    srcp = jnp.pad(src, ((0, 0), (0, pad))).reshape(NW, NCHUNK, CH)
    dstp = jnp.pad(dst, ((0, 0), (0, pad)), constant_values=N).reshape(NW, NCHUNK, CH)

    zeros1 = jnp.zeros((NACC,), jnp.float32)
    zeros_in = jnp.zeros((NACC, IN_CH), jnp.float32)
    zeros_cls = jnp.zeros((NACC, CLS_CH), jnp.float32)

    deg = _sc_degree(dstp, zeros1)
    dega = deg[0, :N].reshape(N, 1)
    degb = deg[1, :N].reshape(N, 1)

    xp, dinv = _tc_prescale(dega, degb, x)

    z1 = _sc_agg_in(srcp, dstp, xp, zeros_in)
    y2 = _tc_mid(z1[0, :N], z1[1, :N], xp, dinv, W1,
                 b1.reshape(1, HID_CH), W2)

    z2 = _sc_agg_cls(srcp, dstp, y2, zeros_cls)
    out = _tc_final(z2[0, :N], z2[1, :N], y2, dinv, b2.reshape(1, CLS_CH))
    return out


# speedup vs baseline: 1.0106x; 1.0106x over previous
import functools

import jax
import jax.numpy as jnp
from jax import lax
from jax.experimental import pallas as pl
from jax.experimental.pallas import tpu as pltpu
from jax.experimental.pallas import tpu_sc as plsc

N = 10000
E = 320000
IN_CH, HID_CH, CLS_CH = 128, 256, 64

NC, NS = 2, 16
NW = NC * NS
EPW = E // NW
NACC = 10240
RPT = NACC // NS

_mesh = plsc.VectorSubcoreMesh(core_axis_name="c", subcore_axis_name="s")


HCH = 128
HNCHUNK = 80
HEPW_PAD = HNCHUNK * HCH


@functools.partial(
    pl.kernel,
    out_type=jax.ShapeDtypeStruct((NC, NACC), jnp.float32),
    mesh=_mesh,
    scratch_types=[
        pltpu.VMEM((HNCHUNK, HCH), jnp.int32),
        pltpu.VMEM((HCH,), jnp.float32),
        pltpu.VMEM_SHARED((NACC,), jnp.float32),
    ],
)
def _sc_degree(dst_hbm, zeros_hbm, deg_hbm, idx_d, ones_v, deg_sh):
    c = lax.axis_index("c")
    s = lax.axis_index("s")
    w = s * NC + c
    pltpu.sync_copy(dst_hbm.at[w], idx_d)
    for i in range(HCH // 16):
        ones_v[pl.ds(i * 16, 16)] = jnp.ones((16,), jnp.float32)
    r0 = s * RPT
    pltpu.sync_copy(zeros_hbm.at[pl.ds(r0, RPT)], deg_sh.at[pl.ds(r0, RPT)])
    plsc.subcore_barrier()

    def body(j, carry):
        pltpu.sync_copy(ones_v, deg_sh.at[idx_d.at[j]], add=True)
        return carry

    lax.fori_loop(0, HNCHUNK, body, 0)
    plsc.subcore_barrier()
    pltpu.sync_copy(deg_sh.at[pl.ds(r0, RPT)], deg_hbm.at[c, pl.ds(r0, RPT)])


def _make_sc_agg(D, NBUF, CH, NCHUNK):
    params = None if D % 128 == 0 else pltpu.CompilerParams(use_tc_tiling_on_sc=False)
    NPJ = NCHUNK // NBUF
    assert NPJ * NBUF == NCHUNK and CH % 8 == 0 and CH <= 128

    @functools.partial(
        pl.kernel,
        out_type=jax.ShapeDtypeStruct((NC, NACC, D), jnp.float32),
        mesh=_mesh,
        compiler_params=params,
        scratch_types=[
            pltpu.VMEM((NCHUNK, CH), jnp.int32),
            pltpu.VMEM((NCHUNK, CH), jnp.int32),
            pltpu.VMEM_SHARED((NACC, D), jnp.float32),
        ] + [pltpu.VMEM((CH, D), jnp.float32) for _ in range(NBUF)]
          + [pltpu.SemaphoreType.DMA for _ in range(2 * NBUF)],
    )
    def _sc_agg(src_hbm, dst_hbm, feat_hbm, zeros_hbm, out_hbm,
                idx_s, idx_d, acc_sh, *bufs_and_sems):
        rows = bufs_and_sems[:NBUF]
        gsem = bufs_and_sems[NBUF:2 * NBUF]
        ssem = bufs_and_sems[2 * NBUF:]
        c = lax.axis_index("c")
        s = lax.axis_index("s")
        w = s * NC + c
        r0 = s * RPT
        pltpu.sync_copy(src_hbm.at[w], idx_s)
        pltpu.sync_copy(dst_hbm.at[w], idx_d)
        pltpu.sync_copy(zeros_hbm.at[pl.ds(r0, RPT)], acc_sh.at[pl.ds(r0, RPT)])
        plsc.subcore_barrier()

        for k in range(NBUF):
            pltpu.async_copy(feat_hbm.at[idx_s.at[k]], rows[k], gsem[k])

        def body(j, carry):
            for k in range(NBUF):
                ck = j * NBUF + k
                pltpu.make_async_copy(feat_hbm.at[idx_s.at[ck]], rows[k],
                                      gsem[k]).wait()
                pltpu.async_copy(rows[k], acc_sh.at[idx_d.at[ck]], ssem[k],
                                 add=True)
            @pl.when(j < NPJ - 1)
            def _():
                for k in range(NBUF):
                    ck = j * NBUF + k
                    pltpu.make_async_copy(rows[k], acc_sh.at[idx_d.at[ck]],
                                          ssem[k]).wait()
                    pltpu.async_copy(feat_hbm.at[idx_s.at[ck + NBUF]], rows[k],
                                     gsem[k])
            return carry

        lax.fori_loop(0, NPJ, body, 0)
        for k in range(NBUF):
            ck = NCHUNK - NBUF + k
            pltpu.make_async_copy(rows[k], acc_sh.at[idx_d.at[ck]],
                                  ssem[k]).wait()
        plsc.subcore_barrier()
        pltpu.sync_copy(acc_sh.at[pl.ds(r0, RPT)], out_hbm.at[c, pl.ds(r0, RPT)])

    return _sc_agg


CH2, NCHUNK2 = 128, 80
_sc_agg_cls = _make_sc_agg(CLS_CH, 4, CH2, NCHUNK2)


_BR = 1000


def _tc_prescale_body(dega, degb, x, xp, dinv):
    di = lax.rsqrt(dega[...] + degb[...] + 1.0)
    dinv[...] = di
    xp[...] = x[...] * di


def _tc_prescale(dega, degb, x):
    grid = (N // _BR,)
    return pl.pallas_call(
        _tc_prescale_body,
        grid=grid,
        in_specs=[
            pl.BlockSpec((_BR, 1), lambda i: (i, 0)),
            pl.BlockSpec((_BR, 1), lambda i: (i, 0)),
            pl.BlockSpec((_BR, IN_CH), lambda i: (i, 0)),
        ],
        out_specs=[
            pl.BlockSpec((_BR, IN_CH), lambda i: (i, 0)),
            pl.BlockSpec((_BR, 1), lambda i: (i, 0)),
        ],
        out_shape=[
            jax.ShapeDtypeStruct((N, IN_CH), jnp.float32),
            jax.ShapeDtypeStruct((N, 1), jnp.float32),
        ],
    )(dega, degb, x)


def _tc_mid_body(h0a, h0b, h1a, h1b, xp, dinv, W1, b1, W2, y2):
    di = dinv[...]
    xpv = xp[...]
    u_lo = h0a[...] + h0b[...] + xpv[:, :IN_CH // 2]
    u_hi = h1a[...] + h1b[...] + xpv[:, IN_CH // 2:]
    u = jnp.concatenate([u_lo, u_hi], axis=1) * di
    h = jnp.dot(u, W1[...], preferred_element_type=jnp.float32) + b1[...]
    h = jnp.maximum(h, 0.0)
    g = jnp.dot(h, W2[...], preferred_element_type=jnp.float32)
    y2[...] = g * di


def _tc_mid(h0a, h0b, h1a, h1b, xp, dinv, W1, b1, W2):
    grid = (N // _BR,)
    half = pl.BlockSpec((_BR, IN_CH // 2), lambda i: (i, 0))
    return pl.pallas_call(
        _tc_mid_body,
        grid=grid,
        in_specs=[
            half, half, half, half,
            pl.BlockSpec((_BR, IN_CH), lambda i: (i, 0)),
            pl.BlockSpec((_BR, 1), lambda i: (i, 0)),
            pl.BlockSpec((IN_CH, HID_CH), lambda i: (0, 0)),
            pl.BlockSpec((1, HID_CH), lambda i: (0, 0)),
            pl.BlockSpec((HID_CH, CLS_CH), lambda i: (0, 0)),
        ],
        out_specs=pl.BlockSpec((_BR, CLS_CH), lambda i: (i, 0)),
        out_shape=jax.ShapeDtypeStruct((N, CLS_CH), jnp.float32),
    )(h0a, h0b, h1a, h1b, xp, dinv, W1, b1, W2)


def _tc_final_body(z2a, z2b, y2, dinv, b2, out):
    out[...] = (z2a[...] + z2b[...] + y2[...]) * dinv[...] + b2[...]


def _tc_final(z2a, z2b, y2, dinv, b2):
    grid = (N // _BR,)
    return pl.pallas_call(
        _tc_final_body,
        grid=grid,
        in_specs=[
            pl.BlockSpec((_BR, CLS_CH), lambda i: (i, 0)),
            pl.BlockSpec((_BR, CLS_CH), lambda i: (i, 0)),
            pl.BlockSpec((_BR, CLS_CH), lambda i: (i, 0)),
            pl.BlockSpec((_BR, 1), lambda i: (i, 0)),
            pl.BlockSpec((1, CLS_CH), lambda i: (0, 0)),
        ],
        out_specs=pl.BlockSpec((_BR, CLS_CH), lambda i: (i, 0)),
        out_shape=jax.ShapeDtypeStruct((N, CLS_CH), jnp.float32),
    )(z2a, z2b, y2, dinv, b2)


def _pad_edges(v, ch, nchunk, fill):
    pad = nchunk * ch - EPW
    return jnp.pad(v, ((0, 0), (0, pad)), constant_values=fill).reshape(
        NW, nchunk, ch)


def kernel(x, edge_index, W1, b1, W2, b2):
    src = edge_index[0].astype(jnp.int32).reshape(NW, EPW)
    dst = edge_index[1].astype(jnp.int32).reshape(NW, EPW)
    srcp = _pad_edges(src, CH2, NCHUNK2, 0)
    dstp = _pad_edges(dst, CH2, NCHUNK2, N)
    srcp_lo = _pad_edges(2 * src, CH2, NCHUNK2, 0)
    srcp_hi = _pad_edges(2 * src + 1, CH2, NCHUNK2, 0)

    zeros1 = jnp.zeros((NACC,), jnp.float32)
    zeros_cls = jnp.zeros((NACC, CLS_CH), jnp.float32)

    deg = _sc_degree(dstp, zeros1)
    dega = deg[0, :N].reshape(N, 1)
    degb = deg[1, :N].reshape(N, 1)

    xp, dinv = _tc_prescale(dega, degb, x)

    xp2 = xp.reshape(2 * N, IN_CH // 2)
    z1h0 = _sc_agg_cls(srcp_lo, dstp, xp2, zeros_cls)
    z1h1 = _sc_agg_cls(srcp_hi, dstp, xp2, zeros_cls)
    y2 = _tc_mid(z1h0[0, :N], z1h0[1, :N], z1h1[0, :N], z1h1[1, :N],
                 xp, dinv, W1, b1.reshape(1, HID_CH), W2)

    z2 = _sc_agg_cls(srcp, dstp, y2, zeros_cls)
    out = _tc_final(z2[0, :N], z2[1, :N], y2, dinv, b2.reshape(1, CLS_CH))
    return out


# speedup vs baseline: 1.5537x; 1.5374x over previous
import functools

import jax
import jax.numpy as jnp
from jax import lax
from jax.experimental import pallas as pl
from jax.experimental.pallas import tpu as pltpu
from jax.experimental.pallas import tpu_sc as plsc

N = 10000
E = 320000
IN_CH, HID_CH, CLS_CH = 128, 256, 64

NC, NS = 2, 16
NW = NC * NS
EPW = E // NW
NACC = 10240
RPT = NACC // NS

_mesh = plsc.VectorSubcoreMesh(core_axis_name="c", subcore_axis_name="s")


HCH = 128
HNCHUNK = 80
HEPW_PAD = HNCHUNK * HCH


@functools.partial(
    pl.kernel,
    out_type=jax.ShapeDtypeStruct((NC, NACC), jnp.float32),
    mesh=_mesh,
    scratch_types=[
        pltpu.VMEM((HNCHUNK, HCH), jnp.int32),
        pltpu.VMEM((HCH,), jnp.float32),
        pltpu.VMEM_SHARED((NACC,), jnp.float32),
    ],
)
def _sc_degree(dst_hbm, zeros_hbm, deg_hbm, idx_d, ones_v, deg_sh):
    c = lax.axis_index("c")
    s = lax.axis_index("s")
    w = s * NC + c
    pltpu.sync_copy(dst_hbm.at[w], idx_d)
    for i in range(HCH // 16):
        ones_v[pl.ds(i * 16, 16)] = jnp.ones((16,), jnp.float32)
    r0 = s * RPT
    pltpu.sync_copy(zeros_hbm.at[pl.ds(r0, RPT)], deg_sh.at[pl.ds(r0, RPT)])
    plsc.subcore_barrier()

    def body(j, carry):
        pltpu.sync_copy(ones_v, deg_sh.at[idx_d.at[j]], add=True)
        return carry

    lax.fori_loop(0, HNCHUNK, body, 0)
    plsc.subcore_barrier()
    pltpu.sync_copy(deg_sh.at[pl.ds(r0, RPT)], deg_hbm.at[c, pl.ds(r0, RPT)])


CH2, NCHUNK2 = 128, 80
D_AGG = CLS_CH
RPT_T = N // NS


@functools.partial(
    pl.kernel,
    out_type=jax.ShapeDtypeStruct((NC, NACC, D_AGG), jnp.float32),
    mesh=_mesh,
    compiler_params=pltpu.CompilerParams(use_tc_tiling_on_sc=False),
    scratch_types=[
        pltpu.VMEM((NCHUNK2, CH2), jnp.int32),
        pltpu.VMEM((NCHUNK2, CH2), jnp.int32),
        pltpu.VMEM_SHARED((N, D_AGG), jnp.float32),
        pltpu.VMEM_SHARED((NACC, D_AGG), jnp.float32),
        pltpu.VMEM((CH2, D_AGG), jnp.float32),
        pltpu.SemaphoreType.DMA,
    ],
)
def _sc_agg_cls(src_hbm, dst_hbm, feat_hbm, zeros_hbm, out_hbm,
                idx_s, idx_d, table_sh, acc_sh, rows, gsem):
    c = lax.axis_index("c")
    s = lax.axis_index("s")
    w = s * NC + c
    r0 = s * RPT
    t0 = s * RPT_T
    pltpu.sync_copy(src_hbm.at[w], idx_s)
    pltpu.sync_copy(dst_hbm.at[w], idx_d)
    pltpu.sync_copy(feat_hbm.at[pl.ds(t0, RPT_T)], table_sh.at[pl.ds(t0, RPT_T)])
    pltpu.sync_copy(zeros_hbm.at[pl.ds(r0, RPT)], acc_sh.at[pl.ds(r0, RPT)])
    plsc.subcore_barrier()

    def body(j, carry):
        pltpu.async_copy(table_sh.at[idx_s.at[j]], rows, gsem).wait()
        pltpu.sync_copy(rows, acc_sh.at[idx_d.at[j]], add=True)
        return carry

    lax.fori_loop(0, NCHUNK2, body, 0)
    plsc.subcore_barrier()
    pltpu.sync_copy(acc_sh.at[pl.ds(r0, RPT)], out_hbm.at[c, pl.ds(r0, RPT)])


_BR = 1000


def _tc_prescale_body(dega, degb, x, xp, xlo, xhi, dinv):
    di = lax.rsqrt(dega[...] + degb[...] + 1.0)
    dinv[...] = di
    xpv = x[...] * di
    xp[...] = xpv
    xlo[...] = xpv[:, :IN_CH // 2]
    xhi[...] = xpv[:, IN_CH // 2:]


def _tc_prescale(dega, degb, x):
    grid = (N // _BR,)
    return pl.pallas_call(
        _tc_prescale_body,
        grid=grid,
        in_specs=[
            pl.BlockSpec((_BR, 1), lambda i: (i, 0)),
            pl.BlockSpec((_BR, 1), lambda i: (i, 0)),
            pl.BlockSpec((_BR, IN_CH), lambda i: (i, 0)),
        ],
        out_specs=[
            pl.BlockSpec((_BR, IN_CH), lambda i: (i, 0)),
            pl.BlockSpec((_BR, IN_CH // 2), lambda i: (i, 0)),
            pl.BlockSpec((_BR, IN_CH // 2), lambda i: (i, 0)),
            pl.BlockSpec((_BR, 1), lambda i: (i, 0)),
        ],
        out_shape=[
            jax.ShapeDtypeStruct((N, IN_CH), jnp.float32),
            jax.ShapeDtypeStruct((N, IN_CH // 2), jnp.float32),
            jax.ShapeDtypeStruct((N, IN_CH // 2), jnp.float32),
            jax.ShapeDtypeStruct((N, 1), jnp.float32),
        ],
    )(dega, degb, x)


def _tc_mid_body(h0a, h0b, h1a, h1b, xp, dinv, W1, b1, W2, y2):
    di = dinv[...]
    xpv = xp[...]
    u_lo = h0a[...] + h0b[...] + xpv[:, :IN_CH // 2]
    u_hi = h1a[...] + h1b[...] + xpv[:, IN_CH // 2:]
    u = jnp.concatenate([u_lo, u_hi], axis=1) * di
    h = jnp.dot(u, W1[...], preferred_element_type=jnp.float32) + b1[...]
    h = jnp.maximum(h, 0.0)
    g = jnp.dot(h, W2[...], preferred_element_type=jnp.float32)
    y2[...] = g * di


def _tc_mid(h0a, h0b, h1a, h1b, xp, dinv, W1, b1, W2):
    grid = (N // _BR,)
    half = pl.BlockSpec((_BR, IN_CH // 2), lambda i: (i, 0))
    return pl.pallas_call(
        _tc_mid_body,
        grid=grid,
        in_specs=[
            half, half, half, half,
            pl.BlockSpec((_BR, IN_CH), lambda i: (i, 0)),
            pl.BlockSpec((_BR, 1), lambda i: (i, 0)),
            pl.BlockSpec((IN_CH, HID_CH), lambda i: (0, 0)),
            pl.BlockSpec((1, HID_CH), lambda i: (0, 0)),
            pl.BlockSpec((HID_CH, CLS_CH), lambda i: (0, 0)),
        ],
        out_specs=pl.BlockSpec((_BR, CLS_CH), lambda i: (i, 0)),
        out_shape=jax.ShapeDtypeStruct((N, CLS_CH), jnp.float32),
    )(h0a, h0b, h1a, h1b, xp, dinv, W1, b1, W2)


def _tc_final_body(z2a, z2b, y2, dinv, b2, out):
    out[...] = (z2a[...] + z2b[...] + y2[...]) * dinv[...] + b2[...]


def _tc_final(z2a, z2b, y2, dinv, b2):
    grid = (N // _BR,)
    return pl.pallas_call(
        _tc_final_body,
        grid=grid,
        in_specs=[
            pl.BlockSpec((_BR, CLS_CH), lambda i: (i, 0)),
            pl.BlockSpec((_BR, CLS_CH), lambda i: (i, 0)),
            pl.BlockSpec((_BR, CLS_CH), lambda i: (i, 0)),
            pl.BlockSpec((_BR, 1), lambda i: (i, 0)),
            pl.BlockSpec((1, CLS_CH), lambda i: (0, 0)),
        ],
        out_specs=pl.BlockSpec((_BR, CLS_CH), lambda i: (i, 0)),
        out_shape=jax.ShapeDtypeStruct((N, CLS_CH), jnp.float32),
    )(z2a, z2b, y2, dinv, b2)


def _pad_edges(v, ch, nchunk, fill):
    pad = nchunk * ch - EPW
    return jnp.pad(v, ((0, 0), (0, pad)), constant_values=fill).reshape(
        NW, nchunk, ch)


def kernel(x, edge_index, W1, b1, W2, b2):
    src = edge_index[0].astype(jnp.int32).reshape(NW, EPW)
    dst = edge_index[1].astype(jnp.int32).reshape(NW, EPW)
    srcp = _pad_edges(src, CH2, NCHUNK2, 0)
    dstp = _pad_edges(dst, CH2, NCHUNK2, N)

    zeros1 = jnp.zeros((NACC,), jnp.float32)
    zeros_cls = jnp.zeros((NACC, CLS_CH), jnp.float32)

    deg = _sc_degree(dstp, zeros1)
    dega = deg[0, :N].reshape(N, 1)
    degb = deg[1, :N].reshape(N, 1)

    xp, xp_lo, xp_hi, dinv = _tc_prescale(dega, degb, x)

    z1h0 = _sc_agg_cls(srcp, dstp, xp_lo, zeros_cls)
    z1h1 = _sc_agg_cls(srcp, dstp, xp_hi, zeros_cls)
    y2 = _tc_mid(z1h0[0, :N], z1h0[1, :N], z1h1[0, :N], z1h1[1, :N],
                 xp, dinv, W1, b1.reshape(1, HID_CH), W2)

    z2 = _sc_agg_cls(srcp, dstp, y2, zeros_cls)
    out = _tc_final(z2[0, :N], z2[1, :N], y2, dinv, b2.reshape(1, CLS_CH))
    return out


# speedup vs baseline: 1.8963x; 1.2206x over previous
import functools

import jax
import jax.numpy as jnp
from jax import lax
from jax.experimental import pallas as pl
from jax.experimental.pallas import tpu as pltpu
from jax.experimental.pallas import tpu_sc as plsc

N = 10000
E = 320000
IN_CH, HID_CH, CLS_CH = 128, 256, 64

NC, NS = 2, 16
NW = NC * NS
EPW = E // NW
NACC = 10240
RPT = NACC // NS

_mesh = plsc.VectorSubcoreMesh(core_axis_name="c", subcore_axis_name="s")


HCH = 128
HNCHUNK = 80
HEPW_PAD = HNCHUNK * HCH


@functools.partial(
    pl.kernel,
    out_type=jax.ShapeDtypeStruct((NC, NACC), jnp.float32),
    mesh=_mesh,
    scratch_types=[
        pltpu.VMEM((HNCHUNK, HCH), jnp.int32),
        pltpu.VMEM((HCH,), jnp.float32),
        pltpu.VMEM_SHARED((NACC,), jnp.float32),
    ],
)
def _sc_degree(dst_hbm, zeros_hbm, deg_hbm, idx_d, ones_v, deg_sh):
    c = lax.axis_index("c")
    s = lax.axis_index("s")
    w = s * NC + c
    pltpu.sync_copy(dst_hbm.at[w], idx_d)
    for i in range(HCH // 16):
        ones_v[pl.ds(i * 16, 16)] = jnp.ones((16,), jnp.float32)
    r0 = s * RPT
    pltpu.sync_copy(zeros_hbm.at[pl.ds(r0, RPT)], deg_sh.at[pl.ds(r0, RPT)])
    plsc.subcore_barrier()

    def body(j, carry):
        pltpu.sync_copy(ones_v, deg_sh.at[idx_d.at[j]], add=True)
        return carry

    lax.fori_loop(0, HNCHUNK, body, 0)
    plsc.subcore_barrier()
    pltpu.sync_copy(deg_sh.at[pl.ds(r0, RPT)], deg_hbm.at[c, pl.ds(r0, RPT)])


CH2, NCHUNK2 = 128, 80
D_AGG = CLS_CH
RPT_T = N // NS


@functools.partial(
    pl.kernel,
    out_type=jax.ShapeDtypeStruct((NC, NACC, D_AGG), jnp.float32),
    mesh=_mesh,
    compiler_params=pltpu.CompilerParams(use_tc_tiling_on_sc=False),
    scratch_types=[
        pltpu.VMEM((NCHUNK2, CH2), jnp.int32),
        pltpu.VMEM((NCHUNK2, CH2), jnp.int32),
        pltpu.VMEM_SHARED((N, D_AGG), jnp.float32),
        pltpu.VMEM_SHARED((NACC, D_AGG), jnp.float32),
        pltpu.VMEM((CH2, D_AGG), jnp.float32),
        pltpu.VMEM((CH2, D_AGG), jnp.float32),
        pltpu.SemaphoreType.DMA,
        pltpu.SemaphoreType.DMA,
        pltpu.SemaphoreType.DMA,
        pltpu.SemaphoreType.DMA,
    ],
)
def _sc_agg_cls(src_hbm, dst_hbm, feat_hbm, zeros_hbm, out_hbm,
                idx_s, idx_d, table_sh, acc_sh, rows0, rows1,
                gsem0, gsem1, ssem0, ssem1):
    c = lax.axis_index("c")
    s = lax.axis_index("s")
    w = s * NC + c
    r0 = s * RPT
    t0 = s * RPT_T
    pltpu.sync_copy(src_hbm.at[w], idx_s)
    pltpu.sync_copy(dst_hbm.at[w], idx_d)
    pltpu.sync_copy(feat_hbm.at[pl.ds(t0, RPT_T)], table_sh.at[pl.ds(t0, RPT_T)])
    pltpu.sync_copy(zeros_hbm.at[pl.ds(r0, RPT)], acc_sh.at[pl.ds(r0, RPT)])
    plsc.subcore_barrier()

    rows = (rows0, rows1)
    gsem = (gsem0, gsem1)
    ssem = (ssem0, ssem1)
    NPJ = NCHUNK2 // 2
    for k in range(2):
        pltpu.async_copy(table_sh.at[idx_s.at[k]], rows[k], gsem[k])

    def body(j, carry):
        for k in range(2):
            ck = 2 * j + k
            pltpu.make_async_copy(table_sh.at[idx_s.at[ck]], rows[k],
                                  gsem[k]).wait()
            pltpu.async_copy(rows[k], acc_sh.at[idx_d.at[ck]], ssem[k],
                             add=True)

        @pl.when(j < NPJ - 1)
        def _():
            for k in range(2):
                ck = 2 * j + k
                pltpu.make_async_copy(rows[k], acc_sh.at[idx_d.at[ck]],
                                      ssem[k]).wait()
                pltpu.async_copy(table_sh.at[idx_s.at[ck + 2]], rows[k], gsem[k])
        return carry

    lax.fori_loop(0, NPJ, body, 0)
    for k in range(2):
        ck = NCHUNK2 - 2 + k
        pltpu.make_async_copy(rows[k], acc_sh.at[idx_d.at[ck]], ssem[k]).wait()
    plsc.subcore_barrier()
    pltpu.sync_copy(acc_sh.at[pl.ds(r0, RPT)], out_hbm.at[c, pl.ds(r0, RPT)])


_BR = 1000


def _tc_prescale_body(dega, degb, x, xp, xlo, xhi, dinv):
    di = lax.rsqrt(dega[...] + degb[...] + 1.0)
    dinv[...] = di
    xpv = x[...] * di
    xp[...] = xpv
    xlo[...] = xpv[:, :IN_CH // 2]
    xhi[...] = xpv[:, IN_CH // 2:]


def _tc_prescale(dega, degb, x):
    grid = (N // _BR,)
    return pl.pallas_call(
        _tc_prescale_body,
        grid=grid,
        in_specs=[
            pl.BlockSpec((_BR, 1), lambda i: (i, 0)),
            pl.BlockSpec((_BR, 1), lambda i: (i, 0)),
            pl.BlockSpec((_BR, IN_CH), lambda i: (i, 0)),
        ],
        out_specs=[
            pl.BlockSpec((_BR, IN_CH), lambda i: (i, 0)),
            pl.BlockSpec((_BR, IN_CH // 2), lambda i: (i, 0)),
            pl.BlockSpec((_BR, IN_CH // 2), lambda i: (i, 0)),
            pl.BlockSpec((_BR, 1), lambda i: (i, 0)),
        ],
        out_shape=[
            jax.ShapeDtypeStruct((N, IN_CH), jnp.float32),
            jax.ShapeDtypeStruct((N, IN_CH // 2), jnp.float32),
            jax.ShapeDtypeStruct((N, IN_CH // 2), jnp.float32),
            jax.ShapeDtypeStruct((N, 1), jnp.float32),
        ],
    )(dega, degb, x)


def _tc_mid_body(h0a, h0b, h1a, h1b, xp, dinv, W1, b1, W2, y2):
    di = dinv[...]
    xpv = xp[...]
    u_lo = h0a[...] + h0b[...] + xpv[:, :IN_CH // 2]
    u_hi = h1a[...] + h1b[...] + xpv[:, IN_CH // 2:]
    u = jnp.concatenate([u_lo, u_hi], axis=1) * di
    h = jnp.dot(u, W1[...], preferred_element_type=jnp.float32) + b1[...]
    h = jnp.maximum(h, 0.0)
    g = jnp.dot(h, W2[...], preferred_element_type=jnp.float32)
    y2[...] = g * di


def _tc_mid(h0a, h0b, h1a, h1b, xp, dinv, W1, b1, W2):
    grid = (N // _BR,)
    half = pl.BlockSpec((_BR, IN_CH // 2), lambda i: (i, 0))
    return pl.pallas_call(
        _tc_mid_body,
        grid=grid,
        in_specs=[
            half, half, half, half,
            pl.BlockSpec((_BR, IN_CH), lambda i: (i, 0)),
            pl.BlockSpec((_BR, 1), lambda i: (i, 0)),
            pl.BlockSpec((IN_CH, HID_CH), lambda i: (0, 0)),
            pl.BlockSpec((1, HID_CH), lambda i: (0, 0)),
            pl.BlockSpec((HID_CH, CLS_CH), lambda i: (0, 0)),
        ],
        out_specs=pl.BlockSpec((_BR, CLS_CH), lambda i: (i, 0)),
        out_shape=jax.ShapeDtypeStruct((N, CLS_CH), jnp.float32),
    )(h0a, h0b, h1a, h1b, xp, dinv, W1, b1, W2)


def _tc_final_body(z2a, z2b, y2, dinv, b2, out):
    out[...] = (z2a[...] + z2b[...] + y2[...]) * dinv[...] + b2[...]


def _tc_final(z2a, z2b, y2, dinv, b2):
    grid = (N // _BR,)
    return pl.pallas_call(
        _tc_final_body,
        grid=grid,
        in_specs=[
            pl.BlockSpec((_BR, CLS_CH), lambda i: (i, 0)),
            pl.BlockSpec((_BR, CLS_CH), lambda i: (i, 0)),
            pl.BlockSpec((_BR, CLS_CH), lambda i: (i, 0)),
            pl.BlockSpec((_BR, 1), lambda i: (i, 0)),
            pl.BlockSpec((1, CLS_CH), lambda i: (0, 0)),
        ],
        out_specs=pl.BlockSpec((_BR, CLS_CH), lambda i: (i, 0)),
        out_shape=jax.ShapeDtypeStruct((N, CLS_CH), jnp.float32),
    )(z2a, z2b, y2, dinv, b2)


def _pad_edges(v, ch, nchunk, fill):
    pad = nchunk * ch - EPW
    return jnp.pad(v, ((0, 0), (0, pad)), constant_values=fill).reshape(
        NW, nchunk, ch)


def kernel(x, edge_index, W1, b1, W2, b2):
    src = edge_index[0].astype(jnp.int32).reshape(NW, EPW)
    dst = edge_index[1].astype(jnp.int32).reshape(NW, EPW)
    srcp = _pad_edges(src, CH2, NCHUNK2, 0)
    dstp = _pad_edges(dst, CH2, NCHUNK2, N)

    zeros1 = jnp.zeros((NACC,), jnp.float32)
    zeros_cls = jnp.zeros((NACC, CLS_CH), jnp.float32)

    deg = _sc_degree(dstp, zeros1)
    dega = deg[0, :N].reshape(N, 1)
    degb = deg[1, :N].reshape(N, 1)

    xp, xp_lo, xp_hi, dinv = _tc_prescale(dega, degb, x)

    z1h0 = _sc_agg_cls(srcp, dstp, xp_lo, zeros_cls)
    z1h1 = _sc_agg_cls(srcp, dstp, xp_hi, zeros_cls)
    y2 = _tc_mid(z1h0[0, :N], z1h0[1, :N], z1h1[0, :N], z1h1[1, :N],
                 xp, dinv, W1, b1.reshape(1, HID_CH), W2)

    z2 = _sc_agg_cls(srcp, dstp, y2, zeros_cls)
    out = _tc_final(z2[0, :N], z2[1, :N], y2, dinv, b2.reshape(1, CLS_CH))
    return out


# speedup vs baseline: 1.8984x; 1.0011x over previous
import functools

import jax
import jax.numpy as jnp
from jax import lax
from jax.experimental import pallas as pl
from jax.experimental.pallas import tpu as pltpu
from jax.experimental.pallas import tpu_sc as plsc

N = 10000
E = 320000
IN_CH, HID_CH, CLS_CH = 128, 256, 64

NC, NS = 2, 16
NW = NC * NS
EPW = E // NW
NACC = 10240
RPT = NACC // NS

_mesh = plsc.VectorSubcoreMesh(core_axis_name="c", subcore_axis_name="s")


HCH = 128
HNCHUNK = 80
HEPW_PAD = HNCHUNK * HCH


@functools.partial(
    pl.kernel,
    out_type=jax.ShapeDtypeStruct((NC, NACC), jnp.float32),
    mesh=_mesh,
    scratch_types=[
        pltpu.VMEM((HNCHUNK, HCH), jnp.int32),
        pltpu.VMEM((HCH,), jnp.float32),
        pltpu.VMEM_SHARED((NACC,), jnp.float32),
    ],
)
def _sc_degree(dst_hbm, zeros_hbm, deg_hbm, idx_d, ones_v, deg_sh):
    c = lax.axis_index("c")
    s = lax.axis_index("s")
    w = s * NC + c
    pltpu.sync_copy(dst_hbm.at[w], idx_d)
    for i in range(HCH // 16):
        ones_v[pl.ds(i * 16, 16)] = jnp.ones((16,), jnp.float32)
    r0 = s * RPT
    pltpu.sync_copy(zeros_hbm.at[pl.ds(r0, RPT)], deg_sh.at[pl.ds(r0, RPT)])
    plsc.subcore_barrier()

    def body(j, carry):
        pltpu.sync_copy(ones_v, deg_sh.at[idx_d.at[j]], add=True)
        return carry

    lax.fori_loop(0, HNCHUNK, body, 0)
    plsc.subcore_barrier()
    pltpu.sync_copy(deg_sh.at[pl.ds(r0, RPT)], deg_hbm.at[c, pl.ds(r0, RPT)])


CH2, NCHUNK2 = 128, 80
D_AGG = CLS_CH
RPT_T = N // NS


def _make_sc_agg(n_phase):
    @functools.partial(
        pl.kernel,
        out_type=jax.ShapeDtypeStruct((n_phase, NC, NACC, D_AGG), jnp.float32),
        mesh=_mesh,
        compiler_params=pltpu.CompilerParams(use_tc_tiling_on_sc=False),
        scratch_types=[
            pltpu.VMEM((NCHUNK2, CH2), jnp.int32),
            pltpu.VMEM((NCHUNK2, CH2), jnp.int32),
            pltpu.VMEM_SHARED((N, D_AGG), jnp.float32),
            pltpu.VMEM_SHARED((NACC, D_AGG), jnp.float32),
            pltpu.VMEM((CH2, D_AGG), jnp.float32),
            pltpu.VMEM((CH2, D_AGG), jnp.float32),
            pltpu.SemaphoreType.DMA,
            pltpu.SemaphoreType.DMA,
            pltpu.SemaphoreType.DMA,
            pltpu.SemaphoreType.DMA,
        ],
    )
    def _sc_agg(src_hbm, dst_hbm, *feats_zeros_out_scratch):
        feats = feats_zeros_out_scratch[:n_phase]
        (zeros_hbm, out_hbm, idx_s, idx_d, table_sh, acc_sh, rows0, rows1,
         gsem0, gsem1, ssem0, ssem1) = feats_zeros_out_scratch[n_phase:]
        c = lax.axis_index("c")
        s = lax.axis_index("s")
        w = s * NC + c
        r0 = s * RPT
        t0 = s * RPT_T
        pltpu.sync_copy(src_hbm.at[w], idx_s)
        pltpu.sync_copy(dst_hbm.at[w], idx_d)

        rows = (rows0, rows1)
        gsem = (gsem0, gsem1)
        ssem = (ssem0, ssem1)
        NPJ = NCHUNK2 // 2

        for h in range(n_phase):
            pltpu.sync_copy(feats[h].at[pl.ds(t0, RPT_T)],
                            table_sh.at[pl.ds(t0, RPT_T)])
            pltpu.sync_copy(zeros_hbm.at[pl.ds(r0, RPT)],
                            acc_sh.at[pl.ds(r0, RPT)])
            plsc.subcore_barrier()

            for k in range(2):
                pltpu.async_copy(table_sh.at[idx_s.at[k]], rows[k], gsem[k])

            def body(j, carry):
                for k in range(2):
                    ck = 2 * j + k
                    pltpu.make_async_copy(table_sh.at[idx_s.at[ck]], rows[k],
                                          gsem[k]).wait()
                    pltpu.async_copy(rows[k], acc_sh.at[idx_d.at[ck]], ssem[k],
                                     add=True)

                @pl.when(j < NPJ - 1)
                def _():
                    for k in range(2):
                        ck = 2 * j + k
                        pltpu.make_async_copy(rows[k], acc_sh.at[idx_d.at[ck]],
                                              ssem[k]).wait()
                        pltpu.async_copy(table_sh.at[idx_s.at[ck + 2]], rows[k],
                                         gsem[k])
                return carry

            lax.fori_loop(0, NPJ, body, 0)
            for k in range(2):
                ck = NCHUNK2 - 2 + k
                pltpu.make_async_copy(rows[k], acc_sh.at[idx_d.at[ck]],
                                      ssem[k]).wait()
            plsc.subcore_barrier()
            pltpu.sync_copy(acc_sh.at[pl.ds(r0, RPT)],
                            out_hbm.at[h, c, pl.ds(r0, RPT)])

    return _sc_agg


_sc_agg1 = _make_sc_agg(1)
_sc_agg2 = _make_sc_agg(2)


_BR = 1000


def _tc_prescale_body(dega, degb, x, xp, xlo, xhi, dinv):
    di = lax.rsqrt(dega[...] + degb[...] + 1.0)
    dinv[...] = di
    xpv = x[...] * di
    xp[...] = xpv
    xlo[...] = xpv[:, :IN_CH // 2]
    xhi[...] = xpv[:, IN_CH // 2:]


def _tc_prescale(dega, degb, x):
    grid = (N // _BR,)
    return pl.pallas_call(
        _tc_prescale_body,
        grid=grid,
        in_specs=[
            pl.BlockSpec((_BR, 1), lambda i: (i, 0)),
            pl.BlockSpec((_BR, 1), lambda i: (i, 0)),
            pl.BlockSpec((_BR, IN_CH), lambda i: (i, 0)),
        ],
        out_specs=[
            pl.BlockSpec((_BR, IN_CH), lambda i: (i, 0)),
            pl.BlockSpec((_BR, IN_CH // 2), lambda i: (i, 0)),
            pl.BlockSpec((_BR, IN_CH // 2), lambda i: (i, 0)),
            pl.BlockSpec((_BR, 1), lambda i: (i, 0)),
        ],
        out_shape=[
            jax.ShapeDtypeStruct((N, IN_CH), jnp.float32),
            jax.ShapeDtypeStruct((N, IN_CH // 2), jnp.float32),
            jax.ShapeDtypeStruct((N, IN_CH // 2), jnp.float32),
            jax.ShapeDtypeStruct((N, 1), jnp.float32),
        ],
    )(dega, degb, x)


def _tc_mid_body(h0a, h0b, h1a, h1b, xp, dinv, W1, b1, W2, y2):
    di = dinv[...]
    xpv = xp[...]
    u_lo = h0a[...] + h0b[...] + xpv[:, :IN_CH // 2]
    u_hi = h1a[...] + h1b[...] + xpv[:, IN_CH // 2:]
    u = jnp.concatenate([u_lo, u_hi], axis=1) * di
    h = jnp.dot(u, W1[...], preferred_element_type=jnp.float32) + b1[...]
    h = jnp.maximum(h, 0.0)
    g = jnp.dot(h, W2[...], preferred_element_type=jnp.float32)
    y2[...] = g * di


def _tc_mid(h0a, h0b, h1a, h1b, xp, dinv, W1, b1, W2):
    grid = (N // _BR,)
    half = pl.BlockSpec((_BR, IN_CH // 2), lambda i: (i, 0))
    return pl.pallas_call(
        _tc_mid_body,
        grid=grid,
        in_specs=[
            half, half, half, half,
            pl.BlockSpec((_BR, IN_CH), lambda i: (i, 0)),
            pl.BlockSpec((_BR, 1), lambda i: (i, 0)),
            pl.BlockSpec((IN_CH, HID_CH), lambda i: (0, 0)),
            pl.BlockSpec((1, HID_CH), lambda i: (0, 0)),
            pl.BlockSpec((HID_CH, CLS_CH), lambda i: (0, 0)),
        ],
        out_specs=pl.BlockSpec((_BR, CLS_CH), lambda i: (i, 0)),
        out_shape=jax.ShapeDtypeStruct((N, CLS_CH), jnp.float32),
    )(h0a, h0b, h1a, h1b, xp, dinv, W1, b1, W2)


def _tc_final_body(z2a, z2b, y2, dinv, b2, out):
    out[...] = (z2a[...] + z2b[...] + y2[...]) * dinv[...] + b2[...]


def _tc_final(z2a, z2b, y2, dinv, b2):
    grid = (N // _BR,)
    return pl.pallas_call(
        _tc_final_body,
        grid=grid,
        in_specs=[
            pl.BlockSpec((_BR, CLS_CH), lambda i: (i, 0)),
            pl.BlockSpec((_BR, CLS_CH), lambda i: (i, 0)),
            pl.BlockSpec((_BR, CLS_CH), lambda i: (i, 0)),
            pl.BlockSpec((_BR, 1), lambda i: (i, 0)),
            pl.BlockSpec((1, CLS_CH), lambda i: (0, 0)),
        ],
        out_specs=pl.BlockSpec((_BR, CLS_CH), lambda i: (i, 0)),
        out_shape=jax.ShapeDtypeStruct((N, CLS_CH), jnp.float32),
    )(z2a, z2b, y2, dinv, b2)


def _pad_edges(v, ch, nchunk, fill):
    pad = nchunk * ch - EPW
    return jnp.pad(v, ((0, 0), (0, pad)), constant_values=fill).reshape(
        NW, nchunk, ch)


def kernel(x, edge_index, W1, b1, W2, b2):
    src = edge_index[0].astype(jnp.int32).reshape(NW, EPW)
    dst = edge_index[1].astype(jnp.int32).reshape(NW, EPW)
    srcp = _pad_edges(src, CH2, NCHUNK2, 0)
    dstp = _pad_edges(dst, CH2, NCHUNK2, N)

    zeros1 = jnp.zeros((NACC,), jnp.float32)
    zeros_cls = jnp.zeros((NACC, CLS_CH), jnp.float32)

    deg = _sc_degree(dstp, zeros1)
    dega = deg[0, :N].reshape(N, 1)
    degb = deg[1, :N].reshape(N, 1)

    xp, xp_lo, xp_hi, dinv = _tc_prescale(dega, degb, x)

    z1 = _sc_agg2(srcp, dstp, xp_lo, xp_hi, zeros_cls)
    y2 = _tc_mid(z1[0, 0, :N], z1[0, 1, :N], z1[1, 0, :N], z1[1, 1, :N],
                 xp, dinv, W1, b1.reshape(1, HID_CH), W2)

    z2 = _sc_agg1(srcp, dstp, y2, zeros_cls)
    out = _tc_final(z2[0, 0, :N], z2[0, 1, :N], y2, dinv, b2.reshape(1, CLS_CH))
    return out
